# baseline (speedup 1.0000x reference)
import jax
import jax.numpy as jnp
from jax.experimental import pallas as pl
from jax.experimental.pallas import tpu as pltpu

_BI = 200
_NBUF = 4
_NOUT = 2


def _gcn_manual(seq_ref, w_ref, adj_hbm, out_hbm,
                support_ref, abuf, obuf, asem, osem):
    n = adj_hbm.shape[0]
    nblocks = n // _BI

    def adj_copy(b):
        return pltpu.make_async_copy(
            adj_hbm.at[pl.ds(b * _BI, _BI), :],
            abuf.at[b % _NBUF],
            asem.at[b % _NBUF],
        )

    def out_copy(b):
        return pltpu.make_async_copy(
            obuf.at[b % _NOUT],
            out_hbm.at[pl.ds(b * _BI, _BI), :],
            osem.at[b % _NOUT],
        )

    for b in range(_NBUF):
        adj_copy(b).start()

    support_ref[...] = jnp.dot(
        seq_ref[...], w_ref[...], preferred_element_type=jnp.float32
    )

    for b in range(nblocks):
        adj_copy(b).wait()
        if b >= _NOUT:
            out_copy(b - _NOUT).wait()
        obuf[b % _NOUT] = jnp.tanh(
            jnp.dot(abuf[b % _NBUF], support_ref[...],
                    preferred_element_type=jnp.float32)
        )
        out_copy(b).start()
        if b + _NBUF < nblocks:
            adj_copy(b + _NBUF).start()

    for b in range(nblocks - _NOUT, nblocks):
        out_copy(b).wait()


def kernel(seq, adj, weight):
    n, in_ft = seq.shape
    out_ft = weight.shape[1]
    return pl.pallas_call(
        _gcn_manual,
        in_specs=[
            pl.BlockSpec((n, in_ft), lambda: (0, 0)),
            pl.BlockSpec((in_ft, out_ft), lambda: (0, 0)),
            pl.BlockSpec(memory_space=pl.ANY),
        ],
        out_specs=pl.BlockSpec(memory_space=pl.ANY),
        out_shape=jax.ShapeDtypeStruct((n, out_ft), jnp.float32),
        scratch_shapes=[
            pltpu.VMEM((n, out_ft), jnp.float32),
            pltpu.VMEM((_NBUF, _BI, n), jnp.float32),
            pltpu.VMEM((_NOUT, _BI, out_ft), jnp.float32),
            pltpu.SemaphoreType.DMA((_NBUF,)),
            pltpu.SemaphoreType.DMA((_NOUT,)),
        ],
    )(seq, weight, adj)
